# baseline (speedup 1.0000x reference)
import functools

import jax
import jax.numpy as jnp
from jax import lax
from jax.experimental import pallas as pl
from jax.experimental.pallas import tpu as pltpu
from jax.experimental.pallas import tpu_sc as plsc

N = 10000
F = 128
E = 160000

ROW_BLOCK = 400

SC_CORES = 2
SC_SUBCORES = 16
NWORKERS = SC_CORES * SC_SUBCORES
EDGES_PER_WORKER = E // NWORKERS
CHUNK = 200
NCHUNKS = EDGES_PER_WORKER // CHUNK
NGROUPS = (CHUNK + 15) // 16


def _gc_layer(adj, x, wn, ws, b, *, relu):
    def body(adj_ref, xb_ref, xf_ref, wn_ref, ws_ref, b_ref, out_ref, sup_s):
        @pl.when(pl.program_id(0) == 0)
        def _init():
            sup_s[...] = jnp.dot(xf_ref[...], wn_ref[...],
                                 preferred_element_type=jnp.float32)

        acc = jnp.dot(adj_ref[...], sup_s[...],
                      preferred_element_type=jnp.float32)
        slf = jnp.dot(xb_ref[...], ws_ref[...],
                      preferred_element_type=jnp.float32)
        r = acc + slf + b_ref[...]
        if relu:
            r = jnp.maximum(r, 0.0)
        out_ref[...] = r

    grid = (N // ROW_BLOCK,)
    return pl.pallas_call(
        body,
        grid=grid,
        in_specs=[
            pl.BlockSpec((ROW_BLOCK, N), lambda i: (i, 0)),
            pl.BlockSpec((ROW_BLOCK, F), lambda i: (i, 0)),
            pl.BlockSpec((N, F), lambda i: (0, 0)),
            pl.BlockSpec((F, F), lambda i: (0, 0)),
            pl.BlockSpec((F, F), lambda i: (0, 0)),
            pl.BlockSpec((1, F), lambda i: (0, 0)),
        ],
        out_specs=pl.BlockSpec((ROW_BLOCK, F), lambda i: (i, 0)),
        out_shape=jax.ShapeDtypeStruct((N, F), jnp.float32),
        scratch_shapes=[pltpu.VMEM((N, F), jnp.float32)],
        compiler_params=pltpu.CompilerParams(
            dimension_semantics=("arbitrary",),
            vmem_limit_bytes=64 * 1024 * 1024,
        ),
    )(adj, x, x, wn, ws, b.reshape(1, F))


def _gc_layer3(adj, x, wn, ws, b, wa, wb, fb):
    def body(adj_ref, xb_ref, xf_ref, wn_ref, ws_ref, b_ref,
             wa_ref, wb_ref, fb_ref, z_ref, zx_ref, zy_ref, sup_s):
        @pl.when(pl.program_id(0) == 0)
        def _init():
            sup_s[...] = jnp.dot(xf_ref[...], wn_ref[...],
                                 preferred_element_type=jnp.float32)

        acc = jnp.dot(adj_ref[...], sup_s[...],
                      preferred_element_type=jnp.float32)
        slf = jnp.dot(xb_ref[...], ws_ref[...],
                      preferred_element_type=jnp.float32)
        z = acc + slf + b_ref[...]
        z_ref[...] = z
        zx_ref[...] = jnp.dot(z, wa_ref[...], preferred_element_type=jnp.float32)
        zy_ref[...] = jnp.dot(z, wb_ref[...],
                              preferred_element_type=jnp.float32) + fb_ref[...]

    grid = (N // ROW_BLOCK,)
    blk = pl.BlockSpec((ROW_BLOCK, F), lambda i: (i, 0))
    wspec = pl.BlockSpec((F, F), lambda i: (0, 0))
    return pl.pallas_call(
        body,
        grid=grid,
        in_specs=[
            pl.BlockSpec((ROW_BLOCK, N), lambda i: (i, 0)),
            blk,
            pl.BlockSpec((N, F), lambda i: (0, 0)),
            wspec,
            wspec,
            pl.BlockSpec((1, F), lambda i: (0, 0)),
            wspec,
            wspec,
            pl.BlockSpec((1, F), lambda i: (0, 0)),
        ],
        out_specs=(blk, blk, blk),
        out_shape=(
            jax.ShapeDtypeStruct((N, F), jnp.float32),
            jax.ShapeDtypeStruct((N, F), jnp.float32),
            jax.ShapeDtypeStruct((N, F), jnp.float32),
        ),
        scratch_shapes=[pltpu.VMEM((N, F), jnp.float32)],
        compiler_params=pltpu.CompilerParams(
            dimension_semantics=("arbitrary",),
            vmem_limit_bytes=64 * 1024 * 1024,
        ),
    )(adj, x, x, wn, ws, b.reshape(1, F), wa, wb, fb.reshape(1, F))


def _decode(zx, zy, x_idx, y_idx, w2, b16):
    mesh = plsc.VectorSubcoreMesh(core_axis_name="c", subcore_axis_name="s")

    @functools.partial(
        pl.kernel,
        out_type=jax.ShapeDtypeStruct((E,), jnp.float32),
        mesh=mesh,
        scratch_types=[
            pltpu.VMEM((CHUNK,), jnp.int32),
            pltpu.VMEM((CHUNK,), jnp.int32),
            pltpu.VMEM((CHUNK,), jnp.int32),
            pltpu.VMEM((CHUNK,), jnp.int32),
            pltpu.VMEM((CHUNK, F), jnp.float32),
            pltpu.VMEM((CHUNK, F), jnp.float32),
            pltpu.VMEM((CHUNK, F), jnp.float32),
            pltpu.VMEM((CHUNK, F), jnp.float32),
            pltpu.VMEM((F,), jnp.float32),
            pltpu.VMEM((16,), jnp.float32),
            pltpu.VMEM((16, 16), jnp.float32),
            pltpu.VMEM((CHUNK,), jnp.float32),
            pltpu.SemaphoreType.DMA,
            pltpu.SemaphoreType.DMA,
        ],
    )
    def decode(zx_hbm, zy_hbm, xi_hbm, yi_hbm, w2_hbm, b16_hbm, out_hbm,
               xi0, yi0, xi1, yi1, bx0, by0, bx1, by1,
               w2_v, b16_v, accb, out_v, sem0, sem1):
        wid = lax.axis_index("s") * SC_CORES + lax.axis_index("c")
        base = wid * EDGES_PER_WORKER
        pltpu.sync_copy(w2_hbm, w2_v)
        pltpu.sync_copy(b16_hbm, b16_v)
        lanes = lax.iota(jnp.int32, 16)
        stages = [(jnp.bitwise_xor(lanes, s), (lanes & s) == 0) for s in (1, 2, 4, 8)]
        w2v = [w2_v[pl.ds(j * 16, 16)] for j in range(F // 16)]
        b16r = b16_v[...]
        sets = ((xi0, yi0, bx0, by0, sem0), (xi1, yi1, bx1, by1, sem1))

        def issue(c, st):
            xi, yi, bx, by, sem = st
            cb = base + c * CHUNK
            pltpu.sync_copy(xi_hbm.at[pl.ds(cb, CHUNK)], xi)
            pltpu.sync_copy(yi_hbm.at[pl.ds(cb, CHUNK)], yi)
            pltpu.async_copy(zx_hbm.at[xi], bx, sem)
            pltpu.async_copy(zy_hbm.at[yi], by, sem)

        def drain(st):
            xi, yi, bx, by, sem = st
            pltpu.make_async_copy(zx_hbm.at[xi], bx, sem).wait()
            pltpu.make_async_copy(zy_hbm.at[yi], by, sem).wait()

        def compute(c, st):
            _, _, bx, by, _ = st

            def group_body(g, gcarry):
                s0 = jnp.minimum(g * 16, CHUNK - 16)

                @plsc.parallel_loop(0, 16, step=1, unroll=4)
                def edge_body(t):
                    e = s0 + t
                    acc = jnp.zeros((16,), jnp.float32)
                    for j in range(F // 16):
                        vx = bx[e, pl.ds(j * 16, 16)]
                        vy = by[e, pl.ds(j * 16, 16)]
                        h = jnp.maximum(vx + vy, 0.0)
                        acc = acc + h * w2v[j]
                    accb[t, :] = acc
                stack = []
                for t in range(16):
                    vec, lev = accb[t, :], 0
                    while stack and stack[-1][0] == lev:
                        u = stack.pop()[1]
                        idxv, msk = stages[lev]
                        pu = u.at[idxv].get(mode="promise_in_bounds")
                        pv = vec.at[idxv].get(mode="promise_in_bounds")
                        vec = jnp.where(msk, u + pu, vec + pv)
                        lev += 1
                    stack.append((lev, vec))
                tot = stack[0][1] + b16r
                out_v[pl.ds(s0, 16)] = 1.0 / (1.0 + jnp.exp(-tot))
                return gcarry

            lax.fori_loop(0, NGROUPS, group_body, 0)
            pltpu.sync_copy(out_v, out_hbm.at[pl.ds(base + c * CHUNK, CHUNK)])

        issue(0, sets[0])

        @pl.loop(0, NCHUNKS, step=2)
        def _outer(c0):
            for b in range(2):
                c = c0 + b

                @pl.when(c < NCHUNKS)
                def _chunk():
                    @pl.when(c + 1 < NCHUNKS)
                    def _prefetch():
                        issue(c + 1, sets[1 - b])

                    drain(sets[b])
                    compute(c, sets[b])

    return decode(zx, zy, x_idx, y_idx, w2, b16)


def kernel(inputs, adj, x_idx, y_idx,
           gc1_wn, gc1_ws, gc1_b,
           gc2_wn, gc2_ws, gc2_b,
           gc3_wn, gc3_ws, gc3_b,
           fc1_w, fc1_b, fc2_w, fc2_b):
    h1 = _gc_layer(adj, inputs, gc1_wn, gc1_ws, gc1_b, relu=True)
    h2 = _gc_layer(adj, h1, gc2_wn, gc2_ws, gc2_b, relu=True)
    z, zx, zy = _gc_layer3(adj, h2, gc3_wn, gc3_ws, gc3_b,
                           fc1_w[:F], fc1_w[F:], fc1_b)
    w2 = fc2_w.reshape(F)
    b16 = jnp.broadcast_to(fc2_b, (16,))
    out = _decode(zx, zy, x_idx, y_idx, w2, b16)
    return (out.reshape(E, 1), z)


# speedup vs baseline: 1.0103x; 1.0103x over previous
import functools

import jax
import jax.numpy as jnp
from jax import lax
from jax.experimental import pallas as pl
from jax.experimental.pallas import tpu as pltpu
from jax.experimental.pallas import tpu_sc as plsc

N = 10000
F = 128
E = 160000

ROW_BLOCK = 400

SC_CORES = 2
SC_SUBCORES = 16
NWORKERS = SC_CORES * SC_SUBCORES
EDGES_PER_WORKER = E // NWORKERS
CHUNK = 200
NCHUNKS = EDGES_PER_WORKER // CHUNK
NGROUPS = (CHUNK + 15) // 16


def _gc_layer(adj, x, wn, ws, b, *, relu):
    def body(adj_ref, xb_ref, xf_ref, wn_ref, ws_ref, b_ref, out_ref, sup_s):
        @pl.when(pl.program_id(0) == 0)
        def _init():
            sup_s[...] = jnp.dot(xf_ref[...], wn_ref[...],
                                 preferred_element_type=jnp.float32)

        acc = jnp.dot(adj_ref[...], sup_s[...],
                      preferred_element_type=jnp.float32)
        slf = jnp.dot(xb_ref[...], ws_ref[...],
                      preferred_element_type=jnp.float32)
        r = acc + slf + b_ref[...]
        if relu:
            r = jnp.maximum(r, 0.0)
        out_ref[...] = r

    grid = (N // ROW_BLOCK,)
    return pl.pallas_call(
        body,
        grid=grid,
        in_specs=[
            pl.BlockSpec((ROW_BLOCK, N), lambda i: (i, 0)),
            pl.BlockSpec((ROW_BLOCK, F), lambda i: (i, 0)),
            pl.BlockSpec((N, F), lambda i: (0, 0)),
            pl.BlockSpec((F, F), lambda i: (0, 0)),
            pl.BlockSpec((F, F), lambda i: (0, 0)),
            pl.BlockSpec((1, F), lambda i: (0, 0)),
        ],
        out_specs=pl.BlockSpec((ROW_BLOCK, F), lambda i: (i, 0)),
        out_shape=jax.ShapeDtypeStruct((N, F), jnp.float32),
        scratch_shapes=[pltpu.VMEM((N, F), jnp.float32)],
        compiler_params=pltpu.CompilerParams(
            dimension_semantics=("arbitrary",),
            vmem_limit_bytes=64 * 1024 * 1024,
        ),
    )(adj, x, x, wn, ws, b.reshape(1, F))


def _gc_layer3(adj, x, wn, ws, b, wa, wb, fb):
    def body(adj_ref, xb_ref, xf_ref, wn_ref, ws_ref, b_ref,
             wa_ref, wb_ref, fb_ref, z_ref, zx_ref, zy_ref, sup_s):
        @pl.when(pl.program_id(0) == 0)
        def _init():
            sup_s[...] = jnp.dot(xf_ref[...], wn_ref[...],
                                 preferred_element_type=jnp.float32)

        acc = jnp.dot(adj_ref[...], sup_s[...],
                      preferred_element_type=jnp.float32)
        slf = jnp.dot(xb_ref[...], ws_ref[...],
                      preferred_element_type=jnp.float32)
        z = acc + slf + b_ref[...]
        z_ref[...] = z
        zx_ref[...] = jnp.dot(z, wa_ref[...], preferred_element_type=jnp.float32)
        zy_ref[...] = jnp.dot(z, wb_ref[...],
                              preferred_element_type=jnp.float32) + fb_ref[...]

    grid = (N // ROW_BLOCK,)
    blk = pl.BlockSpec((ROW_BLOCK, F), lambda i: (i, 0))
    wspec = pl.BlockSpec((F, F), lambda i: (0, 0))
    return pl.pallas_call(
        body,
        grid=grid,
        in_specs=[
            pl.BlockSpec((ROW_BLOCK, N), lambda i: (i, 0)),
            blk,
            pl.BlockSpec((N, F), lambda i: (0, 0)),
            wspec,
            wspec,
            pl.BlockSpec((1, F), lambda i: (0, 0)),
            wspec,
            wspec,
            pl.BlockSpec((1, F), lambda i: (0, 0)),
        ],
        out_specs=(blk, blk, blk),
        out_shape=(
            jax.ShapeDtypeStruct((N, F), jnp.float32),
            jax.ShapeDtypeStruct((N, F), jnp.float32),
            jax.ShapeDtypeStruct((N, F), jnp.float32),
        ),
        scratch_shapes=[pltpu.VMEM((N, F), jnp.float32)],
        compiler_params=pltpu.CompilerParams(
            dimension_semantics=("arbitrary",),
            vmem_limit_bytes=64 * 1024 * 1024,
        ),
    )(adj, x, x, wn, ws, b.reshape(1, F), wa, wb, fb.reshape(1, F))


def _decode(zx, zy, x_idx, y_idx, w2, b16):
    mesh = plsc.VectorSubcoreMesh(core_axis_name="c", subcore_axis_name="s")

    @functools.partial(
        pl.kernel,
        out_type=jax.ShapeDtypeStruct((E,), jnp.float32),
        mesh=mesh,
        scratch_types=[
            pltpu.VMEM((CHUNK,), jnp.int32),
            pltpu.VMEM((CHUNK,), jnp.int32),
            pltpu.VMEM((CHUNK,), jnp.int32),
            pltpu.VMEM((CHUNK,), jnp.int32),
            pltpu.VMEM((CHUNK, F), jnp.float32),
            pltpu.VMEM((CHUNK, F), jnp.float32),
            pltpu.VMEM((CHUNK, F), jnp.float32),
            pltpu.VMEM((CHUNK, F), jnp.float32),
            pltpu.VMEM((F,), jnp.float32),
            pltpu.VMEM((16,), jnp.float32),
            pltpu.VMEM((16, 16), jnp.float32),
            pltpu.VMEM((CHUNK,), jnp.float32),
            pltpu.VMEM((CHUNK,), jnp.float32),
            pltpu.SemaphoreType.DMA,
            pltpu.SemaphoreType.DMA,
            pltpu.SemaphoreType.DMA,
            pltpu.SemaphoreType.DMA,
        ],
    )
    def decode(zx_hbm, zy_hbm, xi_hbm, yi_hbm, w2_hbm, b16_hbm, out_hbm,
               xi0, yi0, xi1, yi1, bx0, by0, bx1, by1,
               w2_v, b16_v, accb, out0, out1, sem0, sem1, sem_o0, sem_o1):
        wid = lax.axis_index("s") * SC_CORES + lax.axis_index("c")
        base = wid * EDGES_PER_WORKER
        pltpu.sync_copy(w2_hbm, w2_v)
        pltpu.sync_copy(b16_hbm, b16_v)
        lanes = lax.iota(jnp.int32, 16)
        stages = [(jnp.bitwise_xor(lanes, s), (lanes & s) == 0) for s in (1, 2, 4, 8)]
        w2v = [w2_v[pl.ds(j * 16, 16)] for j in range(F // 16)]
        b16r = b16_v[...]
        sets = ((xi0, yi0, bx0, by0, sem0, out0, sem_o0),
                (xi1, yi1, bx1, by1, sem1, out1, sem_o1))

        def issue(c, st):
            xi, yi, bx, by, sem = st[:5]
            cb = base + c * CHUNK
            pltpu.sync_copy(xi_hbm.at[pl.ds(cb, CHUNK)], xi)
            pltpu.sync_copy(yi_hbm.at[pl.ds(cb, CHUNK)], yi)
            pltpu.async_copy(zx_hbm.at[xi], bx, sem)
            pltpu.async_copy(zy_hbm.at[yi], by, sem)

        def drain(st):
            xi, yi, bx, by, sem = st[:5]
            pltpu.make_async_copy(zx_hbm.at[xi], bx, sem).wait()
            pltpu.make_async_copy(zy_hbm.at[yi], by, sem).wait()

        def compute(c, st):
            _, _, bx, by, _, out_v, sem_o = st
            @pl.when(c >= 2)
            def _drain_out():
                cb2 = base + (c - 2) * CHUNK
                pltpu.make_async_copy(
                    out_v, out_hbm.at[pl.ds(cb2, CHUNK)], sem_o).wait()

            def group_body(g, gcarry):
                s0 = jnp.minimum(g * 16, CHUNK - 16)

                @plsc.parallel_loop(0, 16, step=1, unroll=4)
                def edge_body(t):
                    e = s0 + t
                    acc = jnp.zeros((16,), jnp.float32)
                    for j in range(F // 16):
                        vx = bx[e, pl.ds(j * 16, 16)]
                        vy = by[e, pl.ds(j * 16, 16)]
                        h = jnp.maximum(vx + vy, 0.0)
                        acc = acc + h * w2v[j]
                    accb[t, :] = acc
                stack = []
                for t in range(16):
                    vec, lev = accb[t, :], 0
                    while stack and stack[-1][0] == lev:
                        u = stack.pop()[1]
                        idxv, msk = stages[lev]
                        pu = u.at[idxv].get(mode="promise_in_bounds")
                        pv = vec.at[idxv].get(mode="promise_in_bounds")
                        vec = jnp.where(msk, u + pu, vec + pv)
                        lev += 1
                    stack.append((lev, vec))
                tot = stack[0][1] + b16r
                out_v[pl.ds(s0, 16)] = 1.0 / (1.0 + jnp.exp(-tot))
                return gcarry

            lax.fori_loop(0, NGROUPS, group_body, 0)
            pltpu.async_copy(out_v, out_hbm.at[pl.ds(base + c * CHUNK, CHUNK)],
                             sem_o)

        issue(0, sets[0])

        @pl.loop(0, NCHUNKS, step=2)
        def _outer(c0):
            for b in range(2):
                c = c0 + b

                @pl.when(c < NCHUNKS)
                def _chunk():
                    @pl.when(c + 1 < NCHUNKS)
                    def _prefetch():
                        issue(c + 1, sets[1 - b])

                    drain(sets[b])
                    compute(c, sets[b])

        for c in (NCHUNKS - 2, NCHUNKS - 1):
            st = sets[c % 2]
            pltpu.make_async_copy(
                st[5], out_hbm.at[pl.ds(base + c * CHUNK, CHUNK)], st[6]).wait()

    return decode(zx, zy, x_idx, y_idx, w2, b16)


def kernel(inputs, adj, x_idx, y_idx,
           gc1_wn, gc1_ws, gc1_b,
           gc2_wn, gc2_ws, gc2_b,
           gc3_wn, gc3_ws, gc3_b,
           fc1_w, fc1_b, fc2_w, fc2_b):
    h1 = _gc_layer(adj, inputs, gc1_wn, gc1_ws, gc1_b, relu=True)
    h2 = _gc_layer(adj, h1, gc2_wn, gc2_ws, gc2_b, relu=True)
    z, zx, zy = _gc_layer3(adj, h2, gc3_wn, gc3_ws, gc3_b,
                           fc1_w[:F], fc1_w[F:], fc1_b)
    w2 = fc2_w.reshape(F)
    b16 = jnp.broadcast_to(fc2_b, (16,))
    out = _decode(zx, zy, x_idx, y_idx, w2, b16)
    return (out.reshape(E, 1), z)
